# baseline (device time: 128764 ns/iter reference)
import jax
import jax.numpy as jnp
from jax import lax
from jax.experimental import pallas as pl
from jax.experimental.pallas import tpu as pltpu

N_DEV = 16
M = 768
N = 768
CHUNK = M // N_DEV


def kernel(x, Wg, Wu, Wd):
    def body(x_ref, wg_ref, wu_ref, wd_ref, out_ref,
             acc_ref, send_ref, rs_bufs, send_sem, rs_sems, ag_sems):
        my = lax.axis_index("i")
        right = lax.rem(my + 1, N_DEV)

        gate = jnp.dot(x_ref[:, :], wg_ref[:, :],
                       preferred_element_type=jnp.float32)
        up = jnp.dot(x_ref[:, :], wu_ref[:, :],
                     preferred_element_type=jnp.float32)
        h = gate * (up * jax.nn.sigmoid(up))
        acc_ref[:, :] = jnp.dot(h, wd_ref[:, :],
                                preferred_element_type=jnp.float32)

        for t in range(N_DEV - 1):
            s_idx = lax.rem(my - t + N_DEV, N_DEV)
            if t == 0:
                send_ref[:, :] = acc_ref[pl.ds(s_idx * CHUNK, CHUNK), :]
            else:
                send_ref[:, :] = (acc_ref[pl.ds(s_idx * CHUNK, CHUNK), :]
                                  + rs_bufs[t - 1, :, :])
            rdma = pltpu.make_async_remote_copy(
                src_ref=send_ref,
                dst_ref=rs_bufs.at[t],
                send_sem=send_sem,
                recv_sem=rs_sems.at[t],
                device_id=(right,),
                device_id_type=pl.DeviceIdType.MESH,
            )
            rdma.start()
            rdma.wait()

        red_idx = lax.rem(my + 1, N_DEV)
        out_ref[pl.ds(red_idx * CHUNK, CHUNK), :] = (
            acc_ref[pl.ds(red_idx * CHUNK, CHUNK), :]
            + rs_bufs[N_DEV - 2, :, :]
        )

        for g in range(N_DEV - 1):
            s_idx = lax.rem(my + 1 - g + N_DEV, N_DEV)
            rdma = pltpu.make_async_remote_copy(
                src_ref=out_ref.at[pl.ds(s_idx * CHUNK, CHUNK), :],
                dst_ref=out_ref.at[pl.ds(s_idx * CHUNK, CHUNK), :],
                send_sem=send_sem,
                recv_sem=ag_sems.at[g],
                device_id=(right,),
                device_id_type=pl.DeviceIdType.MESH,
            )
            rdma.start()
            rdma.wait()

    return pl.pallas_call(
        body,
        out_shape=jax.ShapeDtypeStruct((M, N), jnp.float32),
        in_specs=[
            pl.BlockSpec(memory_space=pltpu.VMEM),
            pl.BlockSpec(memory_space=pltpu.VMEM),
            pl.BlockSpec(memory_space=pltpu.VMEM),
            pl.BlockSpec(memory_space=pltpu.VMEM),
        ],
        out_specs=pl.BlockSpec(memory_space=pltpu.VMEM),
        scratch_shapes=[
            pltpu.VMEM((M, N), jnp.float32),
            pltpu.VMEM((CHUNK, N), jnp.float32),
            pltpu.VMEM((N_DEV - 1, CHUNK, N), jnp.float32),
            pltpu.SemaphoreType.DMA,
            pltpu.SemaphoreType.DMA((N_DEV - 1,)),
            pltpu.SemaphoreType.DMA((N_DEV - 1,)),
        ],
        compiler_params=pltpu.CompilerParams(
            vmem_limit_bytes=128 * 1024 * 1024,
        ),
    )(x, Wg, Wu, Wd)


# device time: 44418 ns/iter; 2.8989x vs baseline; 2.8989x over previous
import jax
import jax.numpy as jnp
from jax import lax
from jax.experimental import pallas as pl
from jax.experimental.pallas import tpu as pltpu

N_DEV = 16
M = 768
N = 768
HC = 384


def kernel(x, Wg, Wu, Wd):
    def body(x_ref, wg_ref, wu_ref, wd_ref, out_ref,
             acc_ref, h_ref, xb_ref, wgb_ref, wub_ref, wdb_ref,
             rsA1, rsA2, rsA3, rzA, rsB1, rsB2, rsB3, rzB,
             ag_buf, send_sems, rsA_sems, rsB_sems, agA_sems, agB_sems):
        my = lax.axis_index("i")
        p = lax.rem(my, 4)
        z = lax.div(my, 4)
        bx = jnp.where((p == 1) | (p == 2), 1, 0)
        by = jnp.where(p >= 2, 1, 0)
        bz0 = lax.rem(z, 2)
        bz1 = lax.div(z, 2)

        part_x = z * 4 + jnp.bitwise_xor(p, 1)
        part_y = z * 4 + (3 - p)
        part_z1 = jnp.bitwise_xor(z, 1) * 4 + p
        part_z2 = jnp.bitwise_xor(z, 2) * 4 + p

        barrier_sem = pltpu.get_barrier_semaphore()
        for prt in (part_x, part_y, part_z1, part_z2):
            pl.semaphore_signal(
                barrier_sem, inc=1,
                device_id=(prt,), device_id_type=pl.DeviceIdType.MESH,
            )

        a1 = bx * 384
        a2 = a1 + by * 192
        a3 = a2 + bz0 * 96
        b1 = by * 384
        b2 = b1 + bx * 192
        b3 = b2 + bz0 * 96

        xb_ref[:, :] = x_ref[:, :].astype(jnp.bfloat16)
        wgb_ref[:, :] = wg_ref[:, :].astype(jnp.bfloat16)
        wub_ref[:, :] = wu_ref[:, :].astype(jnp.bfloat16)
        wdb_ref[:, :] = wd_ref[:, :].astype(jnp.bfloat16)

        def compute_h(r0):
            xs = xb_ref[pl.ds(r0, 384), :]
            g = jnp.dot(xs, wgb_ref[:, :], preferred_element_type=jnp.float32)
            u = jnp.dot(xs, wub_ref[:, :], preferred_element_type=jnp.float32)
            h_ref[pl.ds(r0, 384), :] = (
                g * (u * jax.nn.sigmoid(u))).astype(jnp.bfloat16)

        def down(r0, L, c0):
            acc_ref[pl.ds(r0, L), pl.ds(c0, HC)] = jnp.dot(
                h_ref[pl.ds(r0, L), :], wdb_ref[:, pl.ds(c0, HC)],
                preferred_element_type=jnp.float32).astype(jnp.bfloat16)

        def xchg(src, dst, ssem_i, rsem_arr, rsem_i, partner):
            return pltpu.make_async_remote_copy(
                src_ref=src, dst_ref=dst,
                send_sem=send_sems.at[ssem_i],
                recv_sem=rsem_arr.at[rsem_i],
                device_id=(partner,),
                device_id_type=pl.DeviceIdType.MESH,
            )

        def addseg(r0, L, c0, buf, rb0):
            acc_ref[pl.ds(r0, L), pl.ds(c0, HC)] = (
                acc_ref[pl.ds(r0, L), pl.ds(c0, HC)]
                + buf[pl.ds(rb0, L), :])

        sA1 = (1 - bx) * 384
        sB1 = (1 - by) * 384
        compute_h(sA1)
        down(sA1, 384, 0)
        pl.semaphore_wait(barrier_sem, 4)
        rA = xchg(acc_ref.at[pl.ds(sA1, 384), pl.ds(0, HC)], rsA1,
                  0, rsA_sems, 0, part_x)
        rA.start()
        compute_h(a1)
        down(sB1, 384, HC)
        rB = xchg(acc_ref.at[pl.ds(sB1, 384), pl.ds(HC, HC)], rsB1,
                  1, rsB_sems, 0, part_y)
        rB.start()
        down(a1, 384, 0)
        down(b1, 384, HC)


        sA2 = a1 + (1 - by) * 192
        sB2 = b1 + (1 - bx) * 192
        rA.wait()
        addseg(sA2, 192, 0, rsA1, (1 - by) * 192)
        rA = xchg(acc_ref.at[pl.ds(sA2, 192), pl.ds(0, HC)], rsA2,
                  2, rsA_sems, 1, part_y)
        rA.start()
        addseg(a2, 192, 0, rsA1, by * 192)
        rB.wait()
        addseg(sB2, 192, HC, rsB1, (1 - bx) * 192)
        rB = xchg(acc_ref.at[pl.ds(sB2, 192), pl.ds(HC, HC)], rsB2,
                  3, rsB_sems, 1, part_x)
        rB.start()
        addseg(b2, 192, HC, rsB1, bx * 192)

        sA3 = a2 + (1 - bz0) * 96
        sB3 = b2 + (1 - bz0) * 96
        rA.wait()
        addseg(sA3, 96, 0, rsA2, (1 - bz0) * 96)
        rA = xchg(acc_ref.at[pl.ds(sA3, 96), pl.ds(0, HC)], rsA3,
                  4, rsA_sems, 2, part_z1)
        rA.start()
        addseg(a3, 96, 0, rsA2, bz0 * 96)
        rB.wait()
        addseg(sB3, 96, HC, rsB2, (1 - bz0) * 96)
        rB = xchg(acc_ref.at[pl.ds(sB3, 96), pl.ds(HC, HC)], rsB3,
                  5, rsB_sems, 2, part_z1)
        rB.start()
        addseg(b3, 96, HC, rsB2, bz0 * 96)

        rA.wait()
        addseg(a3, 96, 0, rsA3, 0)
        rA = xchg(acc_ref.at[pl.ds(a3, 96), pl.ds(0, HC)], rzA,
                  6, rsA_sems, 3, part_z2)
        rA.start()
        rB.wait()
        addseg(b3, 96, HC, rsB3, 0)
        rB = xchg(acc_ref.at[pl.ds(b3, 96), pl.ds(HC, HC)], rzB,
                  7, rsB_sems, 3, part_z2)
        rB.start()

        rA.wait()
        ag_buf[pl.ds(a3, 96), pl.ds(0, HC)] = (
            acc_ref[pl.ds(a3, 96), pl.ds(0, HC)] + rzA[:, :])
        rA = xchg(ag_buf.at[pl.ds(a3, 96), pl.ds(0, HC)],
                  ag_buf.at[pl.ds(a3, 96), pl.ds(0, HC)],
                  8, agA_sems, 0, part_z1)
        rA.start()
        rB.wait()
        ag_buf[pl.ds(b3, 96), pl.ds(HC, HC)] = (
            acc_ref[pl.ds(b3, 96), pl.ds(HC, HC)] + rzB[:, :])
        rB = xchg(ag_buf.at[pl.ds(b3, 96), pl.ds(HC, HC)],
                  ag_buf.at[pl.ds(b3, 96), pl.ds(HC, HC)],
                  9, agB_sems, 0, part_z1)
        rB.start()

        rA.wait()
        rA = xchg(ag_buf.at[pl.ds(a2, 192), pl.ds(0, HC)],
                  ag_buf.at[pl.ds(a2, 192), pl.ds(0, HC)],
                  10, agA_sems, 1, part_y)
        rA.start()
        rB.wait()
        rB = xchg(ag_buf.at[pl.ds(b2, 192), pl.ds(HC, HC)],
                  ag_buf.at[pl.ds(b2, 192), pl.ds(HC, HC)],
                  11, agB_sems, 1, part_x)
        rB.start()

        rA.wait()
        rA = xchg(ag_buf.at[pl.ds(a1, 384), pl.ds(0, HC)],
                  ag_buf.at[pl.ds(a1, 384), pl.ds(0, HC)],
                  12, agA_sems, 2, part_x)
        rA.start()
        rB.wait()
        rB = xchg(ag_buf.at[pl.ds(b1, 384), pl.ds(HC, HC)],
                  ag_buf.at[pl.ds(b1, 384), pl.ds(HC, HC)],
                  13, agB_sems, 2, part_y)
        rB.start()
        out_ref[pl.ds(a1, 384), pl.ds(0, HC)] = (
            ag_buf[pl.ds(a1, 384), pl.ds(0, HC)].astype(jnp.float32))
        out_ref[pl.ds(b1, 384), pl.ds(HC, HC)] = (
            ag_buf[pl.ds(b1, 384), pl.ds(HC, HC)].astype(jnp.float32))
        rA.wait()
        out_ref[pl.ds(sA1, 384), pl.ds(0, HC)] = (
            ag_buf[pl.ds(sA1, 384), pl.ds(0, HC)].astype(jnp.float32))
        rB.wait()
        out_ref[pl.ds(sB1, 384), pl.ds(HC, HC)] = (
            ag_buf[pl.ds(sB1, 384), pl.ds(HC, HC)].astype(jnp.float32))

    bf = jnp.bfloat16
    return pl.pallas_call(
        body,
        out_shape=jax.ShapeDtypeStruct((M, N), jnp.float32),
        in_specs=[
            pl.BlockSpec(memory_space=pltpu.VMEM),
            pl.BlockSpec(memory_space=pltpu.VMEM),
            pl.BlockSpec(memory_space=pltpu.VMEM),
            pl.BlockSpec(memory_space=pltpu.VMEM),
        ],
        out_specs=pl.BlockSpec(memory_space=pltpu.VMEM),
        scratch_shapes=[
            pltpu.VMEM((M, N), bf),
            pltpu.VMEM((M, 1536), bf),
            pltpu.VMEM((M, N), bf),
            pltpu.VMEM((M, 1536), bf),
            pltpu.VMEM((M, 1536), bf),
            pltpu.VMEM((1536, N), bf),
            pltpu.VMEM((384, HC), bf), pltpu.VMEM((192, HC), bf),
            pltpu.VMEM((96, HC), bf), pltpu.VMEM((96, HC), bf),
            pltpu.VMEM((384, HC), bf), pltpu.VMEM((192, HC), bf),
            pltpu.VMEM((96, HC), bf), pltpu.VMEM((96, HC), bf),
            pltpu.VMEM((M, N), bf),
            pltpu.SemaphoreType.DMA((14,)),
            pltpu.SemaphoreType.DMA((4,)),
            pltpu.SemaphoreType.DMA((4,)),
            pltpu.SemaphoreType.DMA((3,)),
            pltpu.SemaphoreType.DMA((3,)),
        ],
        compiler_params=pltpu.CompilerParams(
            vmem_limit_bytes=128 * 1024 * 1024,
            collective_id=0,
        ),
    )(x, Wg, Wu, Wd)


# device time: 44182 ns/iter; 2.9144x vs baseline; 1.0053x over previous
import jax
import jax.numpy as jnp
from jax import lax
from jax.experimental import pallas as pl
from jax.experimental.pallas import tpu as pltpu

N_DEV = 16
M = 768
N = 768
HC = 384


def kernel(x, Wg, Wu, Wd):
    def body(x_ref, wg_ref, wu_ref, wd_ref, out_ref,
             acc_ref, h_ref, xb_ref, wgb_ref, wub_ref, wdb_ref,
             rsA1, rsA2, rsA3, rzA, rsB1, rsB2, rsB3, rzB,
             ag_buf, send_sems, rsA_sems, rsB_sems, agA_sems, agB_sems):
        my = lax.axis_index("i")
        p = lax.rem(my, 4)
        z = lax.div(my, 4)
        bx = jnp.where((p == 1) | (p == 2), 1, 0)
        by = jnp.where(p >= 2, 1, 0)
        bz0 = lax.rem(z, 2)
        bz1 = lax.div(z, 2)

        part_x = z * 4 + jnp.bitwise_xor(p, 1)
        part_y = z * 4 + (3 - p)
        part_z1 = jnp.bitwise_xor(z, 1) * 4 + p
        part_z2 = jnp.bitwise_xor(z, 2) * 4 + p

        barrier_sem = pltpu.get_barrier_semaphore()
        for prt in (part_x, part_y, part_z1, part_z2):
            pl.semaphore_signal(
                barrier_sem, inc=1,
                device_id=(prt,), device_id_type=pl.DeviceIdType.MESH,
            )

        a1 = bx * 384
        a2 = a1 + by * 192
        a3 = a2 + bz0 * 96
        b1 = by * 384
        b2 = b1 + bx * 192
        b3 = b2 + bz0 * 96

        def compute_h(r0):
            xs = xb_ref[pl.ds(r0, 384), :]
            g = jnp.dot(xs, wgb_ref[:, :], preferred_element_type=jnp.float32)
            u = jnp.dot(xs, wub_ref[:, :], preferred_element_type=jnp.float32)
            h_ref[pl.ds(r0, 384), :] = (
                g * (u * jax.nn.sigmoid(u))).astype(jnp.bfloat16)

        def down(r0, L, c0):
            acc_ref[pl.ds(r0, L), pl.ds(c0, HC)] = jnp.dot(
                h_ref[pl.ds(r0, L), :], wdb_ref[:, pl.ds(c0, HC)],
                preferred_element_type=jnp.float32).astype(jnp.bfloat16)

        def xchg(src, dst, ssem_i, rsem_arr, rsem_i, partner):
            return pltpu.make_async_remote_copy(
                src_ref=src, dst_ref=dst,
                send_sem=send_sems.at[ssem_i],
                recv_sem=rsem_arr.at[rsem_i],
                device_id=(partner,),
                device_id_type=pl.DeviceIdType.MESH,
            )

        def addseg(r0, L, c0, buf, rb0):
            acc_ref[pl.ds(r0, L), pl.ds(c0, HC)] = (
                acc_ref[pl.ds(r0, L), pl.ds(c0, HC)]
                + buf[pl.ds(rb0, L), :])

        sA1 = (1 - bx) * 384
        sB1 = (1 - by) * 384
        xb_ref[:, :] = x_ref[:, :].astype(jnp.bfloat16)
        wgb_ref[:, :] = wg_ref[:, :].astype(jnp.bfloat16)
        xs1 = xb_ref[pl.ds(sA1, 384), :]
        g1 = jnp.dot(xs1, wgb_ref[:, :], preferred_element_type=jnp.float32)
        wub_ref[:, :] = wu_ref[:, :].astype(jnp.bfloat16)
        u1 = jnp.dot(xs1, wub_ref[:, :], preferred_element_type=jnp.float32)
        wdb_ref[:, :] = wd_ref[:, :].astype(jnp.bfloat16)
        h_ref[pl.ds(sA1, 384), :] = (
            g1 * (u1 * jax.nn.sigmoid(u1))).astype(jnp.bfloat16)
        down(sA1, 384, 0)
        pl.semaphore_wait(barrier_sem, 4)
        rA = xchg(acc_ref.at[pl.ds(sA1, 384), pl.ds(0, HC)], rsA1,
                  0, rsA_sems, 0, part_x)
        rA.start()
        compute_h(a1)
        down(sB1, 384, HC)
        rB = xchg(acc_ref.at[pl.ds(sB1, 384), pl.ds(HC, HC)], rsB1,
                  1, rsB_sems, 0, part_y)
        rB.start()
        down(a1, 384, 0)
        down(b1, 384, HC)


        sA2 = a1 + (1 - by) * 192
        sB2 = b1 + (1 - bx) * 192
        rA.wait()
        addseg(sA2, 192, 0, rsA1, (1 - by) * 192)
        rA = xchg(acc_ref.at[pl.ds(sA2, 192), pl.ds(0, HC)], rsA2,
                  2, rsA_sems, 1, part_y)
        rA.start()
        addseg(a2, 192, 0, rsA1, by * 192)
        rB.wait()
        addseg(sB2, 192, HC, rsB1, (1 - bx) * 192)
        rB = xchg(acc_ref.at[pl.ds(sB2, 192), pl.ds(HC, HC)], rsB2,
                  3, rsB_sems, 1, part_x)
        rB.start()
        addseg(b2, 192, HC, rsB1, bx * 192)

        sA3 = a2 + (1 - bz0) * 96
        sB3 = b2 + (1 - bz0) * 96
        rA.wait()
        addseg(sA3, 96, 0, rsA2, (1 - bz0) * 96)
        rA = xchg(acc_ref.at[pl.ds(sA3, 96), pl.ds(0, HC)], rsA3,
                  4, rsA_sems, 2, part_z1)
        rA.start()
        addseg(a3, 96, 0, rsA2, bz0 * 96)
        rB.wait()
        addseg(sB3, 96, HC, rsB2, (1 - bz0) * 96)
        rB = xchg(acc_ref.at[pl.ds(sB3, 96), pl.ds(HC, HC)], rsB3,
                  5, rsB_sems, 2, part_z1)
        rB.start()
        addseg(b3, 96, HC, rsB2, bz0 * 96)

        rA.wait()
        addseg(a3, 96, 0, rsA3, 0)
        rA = xchg(acc_ref.at[pl.ds(a3, 96), pl.ds(0, HC)], rzA,
                  6, rsA_sems, 3, part_z2)
        rA.start()
        rB.wait()
        addseg(b3, 96, HC, rsB3, 0)
        rB = xchg(acc_ref.at[pl.ds(b3, 96), pl.ds(HC, HC)], rzB,
                  7, rsB_sems, 3, part_z2)
        rB.start()

        rA.wait()
        ag_buf[pl.ds(a3, 96), pl.ds(0, HC)] = (
            acc_ref[pl.ds(a3, 96), pl.ds(0, HC)] + rzA[:, :])
        rA = xchg(ag_buf.at[pl.ds(a3, 96), pl.ds(0, HC)],
                  ag_buf.at[pl.ds(a3, 96), pl.ds(0, HC)],
                  8, agA_sems, 0, part_z1)
        rA.start()
        rB.wait()
        ag_buf[pl.ds(b3, 96), pl.ds(HC, HC)] = (
            acc_ref[pl.ds(b3, 96), pl.ds(HC, HC)] + rzB[:, :])
        rB = xchg(ag_buf.at[pl.ds(b3, 96), pl.ds(HC, HC)],
                  ag_buf.at[pl.ds(b3, 96), pl.ds(HC, HC)],
                  9, agB_sems, 0, part_z1)
        rB.start()

        rA.wait()
        rA = xchg(ag_buf.at[pl.ds(a2, 192), pl.ds(0, HC)],
                  ag_buf.at[pl.ds(a2, 192), pl.ds(0, HC)],
                  10, agA_sems, 1, part_y)
        rA.start()
        rB.wait()
        rB = xchg(ag_buf.at[pl.ds(b2, 192), pl.ds(HC, HC)],
                  ag_buf.at[pl.ds(b2, 192), pl.ds(HC, HC)],
                  11, agB_sems, 1, part_x)
        rB.start()

        rA.wait()
        rA = xchg(ag_buf.at[pl.ds(a1, 384), pl.ds(0, HC)],
                  ag_buf.at[pl.ds(a1, 384), pl.ds(0, HC)],
                  12, agA_sems, 2, part_x)
        rA.start()
        rB.wait()
        rB = xchg(ag_buf.at[pl.ds(b1, 384), pl.ds(HC, HC)],
                  ag_buf.at[pl.ds(b1, 384), pl.ds(HC, HC)],
                  13, agB_sems, 2, part_y)
        rB.start()
        out_ref[pl.ds(a1, 384), pl.ds(0, HC)] = (
            ag_buf[pl.ds(a1, 384), pl.ds(0, HC)].astype(jnp.float32))
        out_ref[pl.ds(b1, 384), pl.ds(HC, HC)] = (
            ag_buf[pl.ds(b1, 384), pl.ds(HC, HC)].astype(jnp.float32))
        rA.wait()
        out_ref[pl.ds(sA1, 384), pl.ds(0, HC)] = (
            ag_buf[pl.ds(sA1, 384), pl.ds(0, HC)].astype(jnp.float32))
        rB.wait()
        out_ref[pl.ds(sB1, 384), pl.ds(HC, HC)] = (
            ag_buf[pl.ds(sB1, 384), pl.ds(HC, HC)].astype(jnp.float32))

    bf = jnp.bfloat16
    return pl.pallas_call(
        body,
        out_shape=jax.ShapeDtypeStruct((M, N), jnp.float32),
        in_specs=[
            pl.BlockSpec(memory_space=pltpu.VMEM),
            pl.BlockSpec(memory_space=pltpu.VMEM),
            pl.BlockSpec(memory_space=pltpu.VMEM),
            pl.BlockSpec(memory_space=pltpu.VMEM),
        ],
        out_specs=pl.BlockSpec(memory_space=pltpu.VMEM),
        scratch_shapes=[
            pltpu.VMEM((M, N), bf),
            pltpu.VMEM((M, 1536), bf),
            pltpu.VMEM((M, N), bf),
            pltpu.VMEM((M, 1536), bf),
            pltpu.VMEM((M, 1536), bf),
            pltpu.VMEM((1536, N), bf),
            pltpu.VMEM((384, HC), bf), pltpu.VMEM((192, HC), bf),
            pltpu.VMEM((96, HC), bf), pltpu.VMEM((96, HC), bf),
            pltpu.VMEM((384, HC), bf), pltpu.VMEM((192, HC), bf),
            pltpu.VMEM((96, HC), bf), pltpu.VMEM((96, HC), bf),
            pltpu.VMEM((M, N), bf),
            pltpu.SemaphoreType.DMA((14,)),
            pltpu.SemaphoreType.DMA((4,)),
            pltpu.SemaphoreType.DMA((4,)),
            pltpu.SemaphoreType.DMA((3,)),
            pltpu.SemaphoreType.DMA((3,)),
        ],
        compiler_params=pltpu.CompilerParams(
            vmem_limit_bytes=128 * 1024 * 1024,
            collective_id=0,
        ),
    )(x, Wg, Wu, Wd)
